# baseline (speedup 1.0000x reference)
import functools

import jax
import jax.numpy as jnp
from jax import lax
from jax.experimental import pallas as pl
from jax.experimental.pallas import tpu as pltpu
from jax.experimental.pallas import tpu_sc as plsc

N = 8192
K = 8192
D = 256
TOPK_N = 5
PADW = 8

BN = 1024

NC = 2
NS = 16
L = 16
NW = NC * NS
RPW = N // NW
CH = 16
NCH = RPW // CH


def _tc_body(z_ref, e_ref, idx_ref, w_ref, en_ref):
    z = z_ref[...]

    @pl.when(pl.program_id(0) == 0)
    def _():
        e = e_ref[...]
        en_ref[...] = lax.dot_general(jnp.ones((8, D), jnp.float32), e * e,
                                      (((1,), (1,)), ((), ())),
                                      preferred_element_type=jnp.float32,
                                      precision=lax.Precision.HIGHEST)

    en = en_ref[0:1, :]
    zn = jnp.sum(z * z, axis=1, keepdims=True)
    mm = lax.dot_general(z_ref[...].astype(jnp.bfloat16),
                         e_ref[...].astype(jnp.bfloat16),
                         (((1,), (1,)), ((), ())),
                         preferred_element_type=jnp.float32)
    s = (zn + en) - 2.0 * mm
    colf = lax.broadcasted_iota(jnp.int32, s.shape, 1).astype(jnp.float32)
    vals, idxs = [], []
    for j in range(TOPK_N):
        m = jnp.min(s, axis=1, keepdims=True)
        imf = jnp.min(jnp.where(s == m, colf, jnp.float32(K)),
                      axis=1, keepdims=True)
        vals.append(m)
        idxs.append(imf)
        if j < TOPK_N - 1:
            s = jnp.where(colf == imf, jnp.float32(jnp.inf), s)
    v = jnp.concatenate(vals, axis=1)
    ii = jnp.concatenate(idxs, axis=1).astype(jnp.int32)
    wu = jnp.exp(v[:, 0:1] - v)
    w = wu / jnp.sum(wu, axis=1, keepdims=True)
    idx_ref[...] = ii
    w_ref[...] = jnp.concatenate(
        [w, jnp.zeros((BN, PADW - TOPK_N), jnp.float32)], axis=1)


def _tc_topk(z, e):
    n = z.shape[0]
    return pl.pallas_call(
        _tc_body,
        grid=(n // BN,),
        in_specs=[
            pl.BlockSpec((BN, D), lambda i: (i, 0)),
            pl.BlockSpec((K, D), lambda i: (0, 0)),
        ],
        out_specs=[
            pl.BlockSpec((BN, TOPK_N), lambda i: (i, 0)),
            pl.BlockSpec((BN, PADW), lambda i: (i, 0)),
        ],
        out_shape=[
            jax.ShapeDtypeStruct((n, TOPK_N), jnp.int32),
            jax.ShapeDtypeStruct((n, PADW), jnp.float32),
        ],
        scratch_shapes=[pltpu.VMEM((8, K), jnp.float32)],
    )(z, e)


NBUF = 4


def _make_sc_body(rpw, nch):
    def _sc_gather_body(idx_hbm, w_hbm, e_hbm, out_hbm, idx_v, w_v, rows_v,
                        out_v, s0, s1, s2, s3):
        sems = (s0, s1, s2, s3)
        wid = lax.axis_index("s") * NC + lax.axis_index("c")
        pltpu.sync_copy(idx_hbm.at[pl.ds(wid * nch, nch)], idx_v)
        pltpu.sync_copy(w_hbm.at[pl.ds(wid * rpw * PADW, rpw * PADW)],
                        w_v.at[pl.ds(0, rpw * PADW)])

        for b in range(NBUF):
            pltpu.async_copy(e_hbm.at[idx_v.at[b]], rows_v.at[b], sems[b])

        def group_body(g, carry):
            for b in range(NBUF):
                c = g * NBUF + b
                pltpu.make_async_copy(e_hbm.at[idx_v.at[c]], rows_v.at[b],
                                      sems[b]).wait()

                def row_body(r, carry2):
                    wv = w_v[pl.ds(c * CH * PADW + r * PADW, L)]
                    for dv in range(D // L):
                        acc = jnp.zeros((L,), jnp.float32)
                        for j in range(TOPK_N):
                            acc = acc + wv[j] * rows_v[b, r * TOPK_N + j,
                                                       pl.ds(dv * L, L)]
                        out_v[r, pl.ds(dv * L, L)] = acc
                    return carry2

                lax.fori_loop(0, CH, row_body, 0)
                pltpu.sync_copy(out_v,
                                out_hbm.at[pl.ds(wid * rpw + c * CH, CH)])

                @pl.when(c + NBUF < nch)
                def _():
                    pltpu.async_copy(e_hbm.at[idx_v.at[c + NBUF]],
                                     rows_v.at[b], sems[b])
            return carry

        lax.fori_loop(0, nch // NBUF, group_body, 0)

    return _sc_gather_body


def _sc_gather(idx2d, w_flat, e, n):
    rpw = n // NW
    nch = rpw // CH
    mesh = plsc.VectorSubcoreMesh(core_axis_name="c", subcore_axis_name="s")
    fn = functools.partial(
        pl.kernel,
        mesh=mesh,
        out_type=jax.ShapeDtypeStruct((n, D), jnp.float32),
        scratch_types=[
            pltpu.VMEM((nch, CH * TOPK_N), jnp.int32),
            pltpu.VMEM((rpw * PADW + L,), jnp.float32),
            pltpu.VMEM((NBUF, CH * TOPK_N, D), jnp.float32),
            pltpu.VMEM((CH, D), jnp.float32),
            pltpu.SemaphoreType.DMA,
            pltpu.SemaphoreType.DMA,
            pltpu.SemaphoreType.DMA,
            pltpu.SemaphoreType.DMA,
        ],
    )(_make_sc_body(rpw, nch))
    return fn(idx2d, w_flat, e)


NSPLIT = 1


def kernel(inputs_flatten, embed):
    h = N // NSPLIT
    qs, eis = [], []
    for p in range(NSPLIT):
        zp = inputs_flatten[p * h:(p + 1) * h]
        idx5, w8 = _tc_topk(zp, embed)
        q = _sc_gather(idx5.reshape(h // CH, CH * TOPK_N), w8.reshape(-1),
                       embed, h)
        qs.append(q)
        eis.append(idx5[:, 0:1])
    quantize = jnp.concatenate(qs, axis=0)
    encoding_indices = jnp.concatenate(eis, axis=0)
    return (quantize, encoding_indices)


# speedup vs baseline: 1.3189x; 1.3189x over previous
import functools

import jax
import jax.numpy as jnp
from jax import lax
from jax.experimental import pallas as pl
from jax.experimental.pallas import tpu as pltpu
from jax.experimental.pallas import tpu_sc as plsc

N = 8192
K = 8192
D = 256
TOPK_N = 5
PADW = 8

BN = 512

NC = 2
NS = 16
L = 16
NW = NC * NS
RPW = N // NW
CH = 16
NCH = RPW // CH


def _tc_body(z_ref, e_ref, idx_ref, w_ref, en_ref):
    z = z_ref[...]

    @pl.when(pl.program_id(0) == 0)
    def _():
        e = e_ref[...]
        en_ref[...] = lax.dot_general(jnp.ones((8, D), jnp.float32), e * e,
                                      (((1,), (1,)), ((), ())),
                                      preferred_element_type=jnp.float32,
                                      precision=lax.Precision.HIGHEST)

    en = en_ref[0:1, :]
    zn = jnp.sum(z * z, axis=1, keepdims=True)
    mm = lax.dot_general(z_ref[...].astype(jnp.bfloat16),
                         e_ref[...].astype(jnp.bfloat16),
                         (((1,), (1,)), ((), ())),
                         preferred_element_type=jnp.float32)
    s = (zn + en) - 2.0 * mm
    colf = lax.broadcasted_iota(jnp.int32, s.shape, 1).astype(jnp.float32)
    vals, idxs = [], []
    for j in range(TOPK_N):
        m = jnp.min(s, axis=1, keepdims=True)
        imf = jnp.min(jnp.where(s == m, colf, jnp.float32(K)),
                      axis=1, keepdims=True)
        vals.append(m)
        idxs.append(imf)
        if j < TOPK_N - 1:
            s = jnp.where(colf == imf, jnp.float32(jnp.inf), s)
    v = jnp.concatenate(vals, axis=1)
    ii = jnp.concatenate(idxs, axis=1).astype(jnp.int32)
    wu = jnp.exp(v[:, 0:1] - v)
    w = wu / jnp.sum(wu, axis=1, keepdims=True)
    idx_ref[...] = ii
    w_ref[...] = jnp.concatenate(
        [w, jnp.zeros((BN, PADW - TOPK_N), jnp.float32)], axis=1)


def _tc_topk(z, e):
    n = z.shape[0]
    return pl.pallas_call(
        _tc_body,
        grid=(n // BN,),
        in_specs=[
            pl.BlockSpec((BN, D), lambda i: (i, 0)),
            pl.BlockSpec((K, D), lambda i: (0, 0)),
        ],
        out_specs=[
            pl.BlockSpec((BN, TOPK_N), lambda i: (i, 0)),
            pl.BlockSpec((BN, PADW), lambda i: (i, 0)),
        ],
        out_shape=[
            jax.ShapeDtypeStruct((n, TOPK_N), jnp.int32),
            jax.ShapeDtypeStruct((n, PADW), jnp.float32),
        ],
        scratch_shapes=[pltpu.VMEM((8, K), jnp.float32)],
    )(z, e)


NBUF = 4


def _make_sc_body(rpw, nch):
    def _sc_gather_body(idx_hbm, w_hbm, e_hbm, out_hbm, idx_v, w_v, rows_v,
                        out_v, s0, s1, s2, s3, so0, so1):
        sems = (s0, s1, s2, s3)
        osems = (so0, so1)
        wid = lax.axis_index("s") * NC + lax.axis_index("c")
        pltpu.sync_copy(idx_hbm.at[pl.ds(wid * nch, nch)], idx_v)
        pltpu.sync_copy(w_hbm.at[pl.ds(wid * rpw * PADW, rpw * PADW)],
                        w_v.at[pl.ds(0, rpw * PADW)])

        for b in range(NBUF):
            pltpu.async_copy(e_hbm.at[idx_v.at[b]], rows_v.at[b], sems[b])

        def group_body(g, carry):
            for b in range(NBUF):
                c = g * NBUF + b
                ob = b % 2
                pltpu.make_async_copy(e_hbm.at[idx_v.at[c]], rows_v.at[b],
                                      sems[b]).wait()

                @pl.when(c >= 2)
                def _():
                    pltpu.make_async_copy(
                        out_v.at[ob],
                        out_hbm.at[pl.ds(wid * rpw + (c - 2) * CH, CH)],
                        osems[ob]).wait()

                def row_body(r, carry2):
                    wv = w_v[pl.ds(c * CH * PADW + r * PADW, L)]
                    for dv in range(D // L):
                        acc = jnp.zeros((L,), jnp.float32)
                        for j in range(TOPK_N):
                            acc = acc + wv[j] * rows_v[b, r * TOPK_N + j,
                                                       pl.ds(dv * L, L)]
                        out_v[ob, r, pl.ds(dv * L, L)] = acc
                    return carry2

                lax.fori_loop(0, CH, row_body, 0)
                pltpu.async_copy(out_v.at[ob],
                                 out_hbm.at[pl.ds(wid * rpw + c * CH, CH)],
                                 osems[ob])

                @pl.when(c + NBUF < nch)
                def _():
                    pltpu.async_copy(e_hbm.at[idx_v.at[c + NBUF]],
                                     rows_v.at[b], sems[b])
            return carry

        lax.fori_loop(0, nch // NBUF, group_body, 0)
        for ob in range(2):
            pltpu.make_async_copy(
                out_v.at[ob],
                out_hbm.at[pl.ds(wid * rpw + (nch - 2 + ob) * CH, CH)],
                osems[ob]).wait()

    return _sc_gather_body


def _sc_gather(idx2d, w_flat, e, n):
    rpw = n // NW
    nch = rpw // CH
    mesh = plsc.VectorSubcoreMesh(core_axis_name="c", subcore_axis_name="s")
    fn = functools.partial(
        pl.kernel,
        mesh=mesh,
        out_type=jax.ShapeDtypeStruct((n, D), jnp.float32),
        scratch_types=[
            pltpu.VMEM((nch, CH * TOPK_N), jnp.int32),
            pltpu.VMEM((rpw * PADW + L,), jnp.float32),
            pltpu.VMEM((NBUF, CH * TOPK_N, D), jnp.float32),
            pltpu.VMEM((2, CH, D), jnp.float32),
            pltpu.SemaphoreType.DMA,
            pltpu.SemaphoreType.DMA,
            pltpu.SemaphoreType.DMA,
            pltpu.SemaphoreType.DMA,
            pltpu.SemaphoreType.DMA,
            pltpu.SemaphoreType.DMA,
        ],
    )(_make_sc_body(rpw, nch))
    return fn(idx2d, w_flat, e)


NSPLIT = 1


def kernel(inputs_flatten, embed):
    h = N // NSPLIT
    qs, eis = [], []
    for p in range(NSPLIT):
        zp = inputs_flatten[p * h:(p + 1) * h]
        idx5, w8 = _tc_topk(zp, embed)
        q = _sc_gather(idx5.reshape(h // CH, CH * TOPK_N), w8.reshape(-1),
                       embed, h)
        qs.append(q)
        eis.append(idx5[:, 0:1])
    quantize = jnp.concatenate(qs, axis=0)
    encoding_indices = jnp.concatenate(eis, axis=0)
    return (quantize, encoding_indices)
